# baseline (speedup 1.0000x reference)
import functools

import jax
import jax.numpy as jnp
from jax import lax
from jax.experimental import pallas as pl
from jax.experimental.pallas import tpu as pltpu
from jax.experimental.pallas import tpu_sc as plsc

N = 10000
E = 320000
H = 128
B = 256
NC, NS = 2, 16
NW = NC * NS
EPW = E // NW
K = 40
NB = EPW // K
WCH = 40
NWC = N // WCH
GCH = 80
NGC = N // GCH
ROWS_TC = 2000
GRID_TC = N // ROWS_TC

_mesh = plsc.VectorSubcoreMesh(core_axis_name="c", subcore_axis_name="s")


def _wid():
    return lax.axis_index("c") * NS + lax.axis_index("s")


CW = 128


@functools.partial(
    pl.kernel,
    out_type=(
        jax.ShapeDtypeStruct((N, H), jnp.float32),
        jax.ShapeDtypeStruct((NC, N, CW), jnp.float32),
    ),
    mesh=_mesh,
    scratch_types=(
        pltpu.VMEM_SHARED((N, CW), jnp.float32),
        pltpu.VMEM((GCH,), jnp.int32),
        pltpu.VMEM((GCH, H), jnp.float32),
        pltpu.VMEM((6, K), jnp.int32),
        pltpu.VMEM((K, CW), jnp.float32),
        pltpu.SemaphoreType.DMA,
    ) + (pltpu.SemaphoreType.DMA,) * 12,
)
def _sc_prep(labels, emb, dstd, zrows, orows, x0, cnt16,
             cnt_sh, lidx, rows, didx, onesb, sem, *sems):
    c = lax.axis_index("c")
    s = lax.axis_index("s")
    w = c * NS + s
    isem = sems[0:6]
    ssem = sems[6:12]

    pltpu.sync_copy(orows, onesb)

    for k in range(-(-NWC // NS)):
        cid = s + NS * k

        @pl.when(cid < NWC)
        def _():
            pltpu.sync_copy(zrows, cnt_sh.at[pl.ds(cid * WCH, WCH)])

    for k in range(-(-NGC // NW)):
        cid = w + NW * k

        @pl.when(cid < NGC)
        def _():
            base = cid * GCH
            pltpu.sync_copy(labels.at[pl.ds(base, GCH)], lidx)
            pltpu.async_copy(emb.at[lidx], rows, sem).wait()
            pltpu.sync_copy(rows, x0.at[pl.ds(base, GCH)])

    plsc.subcore_barrier()

    e0 = (c * NS + s) * EPW

    def start_didx(b, slot):
        pltpu.async_copy(dstd.at[pl.ds(e0 + b * K, K)], didx.at[slot],
                         isem[slot])

    def wait_didx(b, slot):
        pltpu.make_async_copy(dstd.at[pl.ds(e0 + b * K, K)], didx.at[slot],
                              isem[slot]).wait()

    def start_scatter(slot):
        pltpu.async_copy(onesb, cnt_sh.at[didx.at[slot]], ssem[slot],
                         add=True)

    def wait_scatter(slot):
        pltpu.make_async_copy(onesb, cnt_sh.at[didx.at[slot]],
                              ssem[slot]).wait()

    start_didx(0, 0)
    start_didx(1, 1)
    start_didx(2, 2)

    @pl.loop(0, NB)
    def _(b):
        for ph in range(6):
            sl = ph
            sl3 = (ph + 3) % 6

            @pl.when(b % 6 == ph)
            def _():
                @pl.when(b >= 3)
                def _():
                    wait_scatter(sl3)

                @pl.when(b + 3 < NB)
                def _():
                    start_didx(b + 3, sl3)

                wait_didx(b, sl)
                start_scatter(sl)

    wait_scatter((NB - 3) % 6)
    wait_scatter((NB - 2) % 6)
    wait_scatter((NB - 1) % 6)
    plsc.subcore_barrier()

    for k in range(-(-NWC // NS)):
        cid = s + NS * k

        @pl.when(cid < NWC)
        def _():
            rr = cid * WCH
            pltpu.sync_copy(cnt_sh.at[pl.ds(rr, WCH)],
                            cnt16.at[c, pl.ds(rr, WCH)])


@functools.partial(
    pl.kernel,
    out_type=jax.ShapeDtypeStruct((NC, N, H), jnp.float32),
    mesh=_mesh,
    scratch_types=(
        pltpu.VMEM_SHARED((N, H), jnp.float32),
        pltpu.VMEM((WCH, H), jnp.float32),
        pltpu.VMEM((7, K), jnp.int32),
        pltpu.VMEM((7, K), jnp.int32),
        pltpu.VMEM((7, K, H), jnp.float32),
    ) + (pltpu.SemaphoreType.DMA,) * 21,
)
def _sc_agg(xp, srcd, dstd, agg_out, agg_sh, zbuf, sidx, didx, rows, *sems):
    c = lax.axis_index("c")
    s = lax.axis_index("s")
    isem = sems[0:7]
    gsem = sems[7:14]
    ssem = sems[14:21]

    @pl.loop(0, WCH)
    def _(r):
        for q in range(H // 16):
            zbuf[r, pl.ds(q * 16, 16)] = jnp.zeros((16,), jnp.float32)

    for k in range(-(-NWC // NS)):
        cid = s + NS * k

        @pl.when(cid < NWC)
        def _():
            pltpu.sync_copy(zbuf, agg_sh.at[pl.ds(cid * WCH, WCH)])

    plsc.subcore_barrier()

    e0 = (c * NS + s) * EPW

    def start_idx(b, slot):
        eb = e0 + b * K
        pltpu.async_copy(srcd.at[pl.ds(eb, K)], sidx.at[slot], isem[slot])
        pltpu.async_copy(dstd.at[pl.ds(eb, K)], didx.at[slot], isem[slot])

    def wait_idx(b, slot):
        eb = e0 + b * K
        pltpu.make_async_copy(srcd.at[pl.ds(eb, K)], sidx.at[slot],
                              isem[slot]).wait()
        pltpu.make_async_copy(dstd.at[pl.ds(eb, K)], didx.at[slot],
                              isem[slot]).wait()

    def start_gather(slot):
        pltpu.async_copy(xp.at[sidx.at[slot]], rows.at[slot], gsem[slot])

    def wait_gather(slot):
        pltpu.make_async_copy(xp.at[sidx.at[slot]], rows.at[slot],
                              gsem[slot]).wait()

    def start_scatter(slot):
        pltpu.async_copy(rows.at[slot], agg_sh.at[didx.at[slot]],
                         ssem[slot], add=True)

    def wait_scatter(slot):
        pltpu.make_async_copy(rows.at[slot], agg_sh.at[didx.at[slot]],
                              ssem[slot]).wait()

    start_idx(0, 0)
    start_idx(1, 1)
    start_idx(2, 2)

    @pl.loop(0, NB)
    def _(b):
        for ph in range(7):
            sl = ph
            sl3 = (ph + 3) % 7
            slm = (ph + 6) % 7

            @pl.when(b % 7 == ph)
            def _():
                @pl.when(b >= 4)
                def _():
                    wait_scatter(sl3)

                @pl.when(b + 3 < NB)
                def _():
                    start_idx(b + 3, sl3)

                wait_idx(b, sl)
                start_gather(sl)

                @pl.when(b >= 1)
                def _():
                    wait_gather(slm)
                    start_scatter(slm)

    lsl = (NB - 1) % 7
    wait_gather(lsl)
    start_scatter(lsl)
    wait_scatter((NB - 4) % 7)
    wait_scatter((NB - 3) % 7)
    wait_scatter((NB - 2) % 7)
    wait_scatter(lsl)
    plsc.subcore_barrier()

    for k in range(-(-NWC // NS)):
        cid = s + NS * k

        @pl.when(cid < NWC)
        def _():
            rr = cid * WCH
            pltpu.sync_copy(agg_sh.at[pl.ds(rr, WCH)],
                            agg_out.at[c, pl.ds(rr, WCH)])


@functools.partial(
    pl.kernel,
    out_type=jax.ShapeDtypeStruct((B, H), jnp.float32),
    mesh=_mesh,
    scratch_types=(
        pltpu.VMEM((B // NW,), jnp.int32),
        pltpu.VMEM((B // NW, H), jnp.float32),
        pltpu.SemaphoreType.DMA,
    ),
)
def _sc_root(x, ridx, out, iv, rows, sem):
    base = _wid() * (B // NW)
    pltpu.sync_copy(ridx.at[pl.ds(base, B // NW)], iv)
    pltpu.async_copy(x.at[iv], rows, sem).wait()
    pltpu.sync_copy(rows, out.at[pl.ds(base, B // NW)])


def _tc_proj_body(x, wp, bp, o):
    acc = lax.dot_general(x[...], wp[...], (((1,), (1,)), ((), ())),
                          preferred_element_type=jnp.float32)
    o[...] = jnp.maximum(acc + bp[...], 0.0)


_tc_proj = pl.pallas_call(
    _tc_proj_body,
    grid=(GRID_TC,),
    in_specs=[
        pl.BlockSpec((ROWS_TC, H), lambda i: (i, 0)),
        pl.BlockSpec((H, H), lambda i: (0, 0)),
        pl.BlockSpec((1, H), lambda i: (0, 0)),
    ],
    out_specs=pl.BlockSpec((ROWS_TC, H), lambda i: (i, 0)),
    out_shape=jax.ShapeDtypeStruct((N, H), jnp.float32),
)


def _tc_rcnt_body(cnt, o):
    s = jnp.sum(cnt[:, :, 0:1], axis=0)
    o[...] = 1.0 / jnp.maximum(s, 1.0)


_tc_rcnt = pl.pallas_call(
    _tc_rcnt_body,
    out_shape=jax.ShapeDtypeStruct((N, 1), jnp.float32),
)


def _make_combine(it):
    def body(agg, x, rcnt, dep, wl, wr, bl, o):
        mean = (agg[0] + agg[1]) * rcnt[...]
        new = lax.dot_general(mean, wl[...], (((1,), (1,)), ((), ())),
                              preferred_element_type=jnp.float32)
        new = new + bl[...]
        new = new + lax.dot_general(x[...], wr[...], (((1,), (1,)), ((), ())),
                                    preferred_element_type=jnp.float32)
        o[...] = jnp.where(dep[...] == it, new, x[...])

    return pl.pallas_call(
        body,
        grid=(GRID_TC,),
        in_specs=[
            pl.BlockSpec((NC, ROWS_TC, H), lambda i: (0, i, 0)),
            pl.BlockSpec((ROWS_TC, H), lambda i: (i, 0)),
            pl.BlockSpec((ROWS_TC, 1), lambda i: (i, 0)),
            pl.BlockSpec((ROWS_TC, 1), lambda i: (i, 0)),
            pl.BlockSpec((H, H), lambda i: (0, 0)),
            pl.BlockSpec((H, H), lambda i: (0, 0)),
            pl.BlockSpec((1, H), lambda i: (0, 0)),
        ],
        out_specs=pl.BlockSpec((ROWS_TC, H), lambda i: (i, 0)),
        out_shape=jax.ShapeDtypeStruct((N, H), jnp.float32),
    )


_tc_combine = [None] + [_make_combine(i) for i in range(1, 6)]


def _impl(node_labels, edges, depths, root_ptrs, emb_table,
          W_proj, b_proj, W_l, b_l, W_r):
    labels = node_labels.reshape(N)
    src = edges[0]
    dst = edges[1]
    dep2 = depths.reshape(N, 1)
    bp = b_proj.reshape(1, H)
    bl = b_l.reshape(1, H)
    ridx = root_ptrs[1:] - 1
    ridx = jnp.where(ridx < 0, ridx + N, ridx).astype(jnp.int32)

    zrows = jnp.zeros((WCH, CW), jnp.float32)
    orows = jnp.ones((K, CW), jnp.float32)
    x, cnt16 = _sc_prep(labels, emb_table, dst, zrows, orows)
    rcnt = _tc_rcnt(cnt16)
    for it in range(1, 6):
        xp = _tc_proj(x, W_proj, bp)
        agg = _sc_agg(xp, src, dst)
        x = _tc_combine[it](agg, x, rcnt, dep2, W_l, W_r, bl)
    return _sc_root(x, ridx)


_impl_jit = jax.jit(_impl)


def kernel(node_labels, edges, depths, root_ptrs, emb_table,
           W_proj, b_proj, W_l, b_l, W_r):
    return _impl_jit(node_labels, edges, depths, root_ptrs, emb_table,
                     W_proj, b_proj, W_l, b_l, W_r)


# speedup vs baseline: 1.0376x; 1.0376x over previous
import functools

import jax
import jax.numpy as jnp
from jax import lax
from jax.experimental import pallas as pl
from jax.experimental.pallas import tpu as pltpu
from jax.experimental.pallas import tpu_sc as plsc

N = 10000
E = 320000
H = 128
B = 256
NC, NS = 2, 16
NW = NC * NS
EPW = E // NW
K = 40
NB = EPW // K
WCH = 40
NWC = N // WCH
GCH = 80
NGC = N // GCH
ROWS_TC = 2000
GRID_TC = N // ROWS_TC

_mesh = plsc.VectorSubcoreMesh(core_axis_name="c", subcore_axis_name="s")


def _wid():
    return lax.axis_index("c") * NS + lax.axis_index("s")


CW = 128


@functools.partial(
    pl.kernel,
    out_type=(
        jax.ShapeDtypeStruct((N, H), jnp.float32),
        jax.ShapeDtypeStruct((NC, N, CW), jnp.float32),
    ),
    mesh=_mesh,
    scratch_types=(
        pltpu.VMEM_SHARED((N, CW), jnp.float32),
        pltpu.VMEM((GCH,), jnp.int32),
        pltpu.VMEM((GCH, H), jnp.float32),
        pltpu.VMEM((6, K), jnp.int32),
        pltpu.VMEM((K, CW), jnp.float32),
        pltpu.SemaphoreType.DMA,
    ) + (pltpu.SemaphoreType.DMA,) * 12,
)
def _sc_prep(labels, emb, dstd, zrows, orows, x0, cnt16,
             cnt_sh, lidx, rows, didx, onesb, sem, *sems):
    c = lax.axis_index("c")
    s = lax.axis_index("s")
    w = c * NS + s
    isem = sems[0:6]
    ssem = sems[6:12]

    pltpu.sync_copy(orows, onesb)

    for k in range(-(-NWC // NS)):
        cid = s + NS * k

        @pl.when(cid < NWC)
        def _():
            pltpu.sync_copy(zrows, cnt_sh.at[pl.ds(cid * WCH, WCH)])

    for k in range(-(-NGC // NW)):
        cid = w + NW * k

        @pl.when(cid < NGC)
        def _():
            base = cid * GCH
            pltpu.sync_copy(labels.at[pl.ds(base, GCH)], lidx)
            pltpu.async_copy(emb.at[lidx], rows, sem).wait()
            pltpu.sync_copy(rows, x0.at[pl.ds(base, GCH)])

    plsc.subcore_barrier()

    e0 = (c * NS + s) * EPW

    def start_didx(b, slot):
        pltpu.async_copy(dstd.at[pl.ds(e0 + b * K, K)], didx.at[slot],
                         isem[slot])

    def wait_didx(b, slot):
        pltpu.make_async_copy(dstd.at[pl.ds(e0 + b * K, K)], didx.at[slot],
                              isem[slot]).wait()

    def start_scatter(slot):
        pltpu.async_copy(onesb, cnt_sh.at[didx.at[slot]], ssem[slot],
                         add=True)

    def wait_scatter(slot):
        pltpu.make_async_copy(onesb, cnt_sh.at[didx.at[slot]],
                              ssem[slot]).wait()

    start_didx(0, 0)
    start_didx(1, 1)
    start_didx(2, 2)

    @pl.loop(0, NB)
    def _(b):
        for ph in range(6):
            sl = ph
            sl3 = (ph + 3) % 6

            @pl.when(b % 6 == ph)
            def _():
                @pl.when(b >= 3)
                def _():
                    wait_scatter(sl3)

                @pl.when(b + 3 < NB)
                def _():
                    start_didx(b + 3, sl3)

                wait_didx(b, sl)
                start_scatter(sl)

    wait_scatter((NB - 3) % 6)
    wait_scatter((NB - 2) % 6)
    wait_scatter((NB - 1) % 6)
    plsc.subcore_barrier()

    for k in range(-(-NWC // NS)):
        cid = s + NS * k

        @pl.when(cid < NWC)
        def _():
            rr = cid * WCH
            pltpu.sync_copy(cnt_sh.at[pl.ds(rr, WCH)],
                            cnt16.at[c, pl.ds(rr, WCH)])


@functools.partial(
    pl.kernel,
    out_type=jax.ShapeDtypeStruct((NC, N, H), jnp.float32),
    mesh=_mesh,
    scratch_types=(
        pltpu.VMEM_SHARED((N, H), jnp.float32),
        pltpu.VMEM((WCH, H), jnp.float32),
        pltpu.VMEM((6, K), jnp.int32),
        pltpu.VMEM((6, K), jnp.int32),
        pltpu.VMEM((6, K, H), jnp.float32),
    ) + (pltpu.SemaphoreType.DMA,) * 18,
)
def _sc_agg(xp, srcd, dstd, agg_out, agg_sh, zbuf, sidx, didx, rows, *sems):
    c = lax.axis_index("c")
    s = lax.axis_index("s")
    isem = sems[0:6]
    gsem = sems[6:12]
    ssem = sems[12:18]

    @pl.loop(0, WCH)
    def _(r):
        for q in range(H // 16):
            zbuf[r, pl.ds(q * 16, 16)] = jnp.zeros((16,), jnp.float32)

    for k in range(-(-NWC // NS)):
        cid = s + NS * k

        @pl.when(cid < NWC)
        def _():
            pltpu.sync_copy(zbuf, agg_sh.at[pl.ds(cid * WCH, WCH)])

    plsc.subcore_barrier()

    e0 = (c * NS + s) * EPW

    def start_idx(b, slot):
        eb = e0 + b * K
        pltpu.async_copy(srcd.at[pl.ds(eb, K)], sidx.at[slot], isem[slot])
        pltpu.async_copy(dstd.at[pl.ds(eb, K)], didx.at[slot], isem[slot])

    def wait_idx(b, slot):
        eb = e0 + b * K
        pltpu.make_async_copy(srcd.at[pl.ds(eb, K)], sidx.at[slot],
                              isem[slot]).wait()
        pltpu.make_async_copy(dstd.at[pl.ds(eb, K)], didx.at[slot],
                              isem[slot]).wait()

    def start_gather(slot):
        pltpu.async_copy(xp.at[sidx.at[slot]], rows.at[slot], gsem[slot])

    def wait_gather(slot):
        pltpu.make_async_copy(xp.at[sidx.at[slot]], rows.at[slot],
                              gsem[slot]).wait()

    def start_scatter(slot):
        pltpu.async_copy(rows.at[slot], agg_sh.at[didx.at[slot]],
                         ssem[slot], add=True)

    def wait_scatter(slot):
        pltpu.make_async_copy(rows.at[slot], agg_sh.at[didx.at[slot]],
                              ssem[slot]).wait()

    start_idx(0, 0)
    start_idx(1, 1)
    start_idx(2, 2)

    @pl.loop(0, NB)
    def _(b):
        for ph in range(6):
            sl = ph
            sl3 = (ph + 3) % 6
            slm = (ph + 5) % 6

            @pl.when(b % 6 == ph)
            def _():
                @pl.when(b >= 3)
                def _():
                    wait_scatter(sl3)

                @pl.when(b + 3 < NB)
                def _():
                    start_idx(b + 3, sl3)

                wait_idx(b, sl)
                start_gather(sl)

                @pl.when(b >= 1)
                def _():
                    wait_gather(slm)
                    start_scatter(slm)

    lsl = (NB - 1) % 6
    wait_gather(lsl)
    start_scatter(lsl)
    wait_scatter((NB - 3) % 6)
    wait_scatter((NB - 2) % 6)
    wait_scatter(lsl)
    plsc.subcore_barrier()

    for k in range(-(-NWC // NS)):
        cid = s + NS * k

        @pl.when(cid < NWC)
        def _():
            rr = cid * WCH
            pltpu.sync_copy(agg_sh.at[pl.ds(rr, WCH)],
                            agg_out.at[c, pl.ds(rr, WCH)])


@functools.partial(
    pl.kernel,
    out_type=jax.ShapeDtypeStruct((B, H), jnp.float32),
    mesh=_mesh,
    scratch_types=(
        pltpu.VMEM((B // NW,), jnp.int32),
        pltpu.VMEM((B // NW, H), jnp.float32),
        pltpu.SemaphoreType.DMA,
    ),
)
def _sc_root(x, ridx, out, iv, rows, sem):
    base = _wid() * (B // NW)
    pltpu.sync_copy(ridx.at[pl.ds(base, B // NW)], iv)
    pltpu.async_copy(x.at[iv], rows, sem).wait()
    pltpu.sync_copy(rows, out.at[pl.ds(base, B // NW)])


def _tc_proj_body(x, wp, bp, o):
    acc = lax.dot_general(x[...], wp[...], (((1,), (1,)), ((), ())),
                          preferred_element_type=jnp.float32)
    o[...] = jnp.maximum(acc + bp[...], 0.0)


_tc_proj = pl.pallas_call(
    _tc_proj_body,
    grid=(GRID_TC,),
    in_specs=[
        pl.BlockSpec((ROWS_TC, H), lambda i: (i, 0)),
        pl.BlockSpec((H, H), lambda i: (0, 0)),
        pl.BlockSpec((1, H), lambda i: (0, 0)),
    ],
    out_specs=pl.BlockSpec((ROWS_TC, H), lambda i: (i, 0)),
    out_shape=jax.ShapeDtypeStruct((N, H), jnp.float32),
)


def _tc_rcnt_body(cnt, o):
    s = jnp.sum(cnt[:, :, 0:1], axis=0)
    o[...] = 1.0 / jnp.maximum(s, 1.0)


_tc_rcnt = pl.pallas_call(
    _tc_rcnt_body,
    out_shape=jax.ShapeDtypeStruct((N, 1), jnp.float32),
)


def _make_combine(it):
    def body(agg, x, rcnt, dep, wl, wr, bl, o):
        mean = (agg[0] + agg[1]) * rcnt[...]
        new = lax.dot_general(mean, wl[...], (((1,), (1,)), ((), ())),
                              preferred_element_type=jnp.float32)
        new = new + bl[...]
        new = new + lax.dot_general(x[...], wr[...], (((1,), (1,)), ((), ())),
                                    preferred_element_type=jnp.float32)
        o[...] = jnp.where(dep[...] == it, new, x[...])

    return pl.pallas_call(
        body,
        grid=(GRID_TC,),
        in_specs=[
            pl.BlockSpec((NC, ROWS_TC, H), lambda i: (0, i, 0)),
            pl.BlockSpec((ROWS_TC, H), lambda i: (i, 0)),
            pl.BlockSpec((ROWS_TC, 1), lambda i: (i, 0)),
            pl.BlockSpec((ROWS_TC, 1), lambda i: (i, 0)),
            pl.BlockSpec((H, H), lambda i: (0, 0)),
            pl.BlockSpec((H, H), lambda i: (0, 0)),
            pl.BlockSpec((1, H), lambda i: (0, 0)),
        ],
        out_specs=pl.BlockSpec((ROWS_TC, H), lambda i: (i, 0)),
        out_shape=jax.ShapeDtypeStruct((N, H), jnp.float32),
    )


_tc_combine = [None] + [_make_combine(i) for i in range(1, 6)]


def _impl(node_labels, edges, depths, root_ptrs, emb_table,
          W_proj, b_proj, W_l, b_l, W_r):
    labels = node_labels.reshape(N)
    src = edges[0]
    dst = edges[1]
    dep2 = depths.reshape(N, 1)
    bp = b_proj.reshape(1, H)
    bl = b_l.reshape(1, H)
    ridx = root_ptrs[1:] - 1
    ridx = jnp.where(ridx < 0, ridx + N, ridx).astype(jnp.int32)

    zrows = jnp.zeros((WCH, CW), jnp.float32)
    orows = jnp.ones((K, CW), jnp.float32)
    x, cnt16 = _sc_prep(labels, emb_table, dst, zrows, orows)
    rcnt = _tc_rcnt(cnt16)
    for it in range(1, 6):
        xp = _tc_proj(x, W_proj, bp)
        agg = _sc_agg(xp, src, dst)
        x = _tc_combine[it](agg, x, rcnt, dep2, W_l, W_r, bl)
    return _sc_root(x, ridx)


_impl_jit = jax.jit(_impl)


def kernel(node_labels, edges, depths, root_ptrs, emb_table,
           W_proj, b_proj, W_l, b_l, W_r):
    return _impl_jit(node_labels, edges, depths, root_ptrs, emb_table,
                     W_proj, b_proj, W_l, b_l, W_r)
